# baseline (speedup 1.0000x reference)
import functools

import jax
import jax.numpy as jnp
from jax import lax
from jax.experimental import pallas as pl
from jax.experimental.pallas import tpu as pltpu
from jax.experimental.pallas import tpu_sc as plsc

B, L, V, D, H, C = 16384, 50, 100000, 128, 256, 1000
LP = 56
NC, NS = 2, 16
NW = NC * NS
SEG_PER_W = B // NW
SEG_BLK = 128
N_BLK = SEG_PER_W // SEG_BLK


def _sc_pool(xpad, emb):
    mesh = plsc.VectorSubcoreMesh(core_axis_name="c", subcore_axis_name="s")

    @functools.partial(
        pl.kernel,
        mesh=mesh,
        out_type=jax.ShapeDtypeStruct((B, D), jnp.float32),
        scratch_types=[
            pltpu.VMEM((SEG_BLK, LP), jnp.int32),
            pltpu.VMEM((LP, D), jnp.float32),
            pltpu.VMEM((SEG_BLK, D), jnp.float32),
            pltpu.SemaphoreType.DMA,
        ],
    )
    def pool(xpad_hbm, emb_hbm, out_hbm, idx_v, gbuf, res_v, sem):
        wid = lax.axis_index("s") * NC + lax.axis_index("c")
        seg0 = wid * SEG_PER_W

        def blk_body(blk, carry):
            base = seg0 + blk * SEG_BLK
            pltpu.sync_copy(xpad_hbm.at[pl.ds(base, SEG_BLK)], idx_v)

            def seg_body(c, carry2):
                pltpu.async_copy(emb_hbm.at[idx_v.at[c]], gbuf, sem).wait()

                def row_body(r, acc):
                    return tuple(acc[j] + gbuf[r, pl.ds(j * 16, 16)]
                                 for j in range(D // 16))

                acc = lax.fori_loop(
                    0, LP, row_body,
                    tuple(jnp.zeros((16,), jnp.float32) for _ in range(D // 16)))
                for j in range(D // 16):
                    res_v[c, pl.ds(j * 16, 16)] = acc[j]
                return carry2

            lax.fori_loop(0, SEG_BLK, seg_body, 0)
            pltpu.sync_copy(res_v, out_hbm.at[pl.ds(base, SEG_BLK)])
            return carry

        lax.fori_loop(0, N_BLK, blk_body, 0)

    return pool(xpad, emb)


def _mlp_body(ps_ref, w1_ref, b1_ref, w2_ref, b2_ref, out_ref):
    h = jnp.dot(ps_ref[...], w1_ref[...], preferred_element_type=jnp.float32)
    h = jnp.maximum(h + b1_ref[...], 0.0)
    logits = jnp.dot(h, w2_ref[...], preferred_element_type=jnp.float32)
    logits = logits + b2_ref[...]
    m = jnp.max(logits, axis=1, keepdims=True)
    lse = jnp.log(jnp.sum(jnp.exp(logits - m), axis=1, keepdims=True)) + m
    out_ref[...] = logits - lse


def _mlp(ps, w1, b1, w2, b2, interpret=False):
    R = 2048
    return pl.pallas_call(
        _mlp_body,
        grid=(B // R,),
        in_specs=[
            pl.BlockSpec((R, D), lambda i: (i, 0)),
            pl.BlockSpec((D, H), lambda i: (0, 0)),
            pl.BlockSpec((1, H), lambda i: (0, 0)),
            pl.BlockSpec((H, C), lambda i: (0, 0)),
            pl.BlockSpec((1, C), lambda i: (0, 0)),
        ],
        out_specs=pl.BlockSpec((R, C), lambda i: (i, 0)),
        out_shape=jax.ShapeDtypeStruct((B, C), jnp.float32),
        interpret=interpret,
    )(ps, w1, b1.reshape(1, H), w2, b2.reshape(1, C))


def kernel(x, emb, w1, b1, w2, b2):
    xpad = jnp.pad(x.astype(jnp.int32), ((0, 0), (0, LP - L)))
    pooled_sum = _sc_pool(xpad, emb)
    return _mlp(pooled_sum, w1 * (1.0 / L), b1, w2, b2)
